# baseline (speedup 1.0000x reference)
import functools

import jax
import jax.numpy as jnp
from jax import lax
from jax.experimental import pallas as pl
from jax.experimental.pallas import tpu as pltpu
from jax.experimental.pallas import tpu_sc as plsc

N = 10000
D = 128
DW = 8
E = 320000
NC = 2
NS = 16
CH = 128
EP = 327680
EPT = EP // NS
NCH = EPT // CH
NR = 4
NP = 10240
RPT = NP // NS

_mesh = plsc.VectorSubcoreMesh(core_axis_name="c", subcore_axis_name="s")

_CONTRACT_LAST = (((1,), (1,)), ((), ()))


def _direction(table, edges, zrs, zrsd, ones, out, outd, cid, sid,
               src_v, dst_v, rows, ones_v, acc, accd,
               sem_g, sem_s, sem_d, deg):
    pltpu.sync_copy(edges.at[0, sid], src_v)
    pltpu.sync_copy(edges.at[1, sid], dst_v)
    r0 = sid * RPT
    pltpu.sync_copy(zrs.at[pl.ds(r0, RPT)], acc.at[pl.ds(r0, RPT)])
    if deg:
        pltpu.sync_copy(zrsd.at[pl.ds(r0, RPT)], accd.at[pl.ds(r0, RPT)])
        pltpu.sync_copy(ones, ones_v)
    plsc.subcore_barrier()

    def gather(c, u):
        pltpu.async_copy(table.at[src_v.at[c]], rows[u], sem_g[u])

    def drain_gather(u):
        pltpu.make_async_copy(table.at[src_v.at[0]], rows[u],
                              sem_g[u]).wait()

    def scatter(c, u):
        pltpu.async_copy(rows[u], acc.at[dst_v.at[c]], sem_s[u],
                         add=True)
        if deg:
            pltpu.async_copy(ones_v, accd.at[dst_v.at[c]], sem_d, add=True)

    def drain_scatter(u):
        pltpu.make_async_copy(rows[u], acc.at[dst_v.at[0]],
                              sem_s[u]).wait()

    for u in range(NR - 1):
        gather(u, u)

    def step(j, carry):
        for u in range(NR):
            c = j * NR + u
            v = (u + NR - 1) % NR
            drain_gather(u)
            scatter(c, u)

            @pl.when(c + NR - 1 < NCH)
            def _():
                @pl.when(c > 0)
                def _():
                    drain_scatter(v)
                gather(c + NR - 1, v)

        return carry

    lax.fori_loop(0, NCH // NR, step, 0)

    for u in range(NR):
        drain_scatter(u)
    if deg:
        def drain_deg(i, carry):
            pltpu.make_async_copy(ones_v, accd.at[dst_v.at[0]], sem_d).wait()
            return carry
        lax.fori_loop(0, NCH, drain_deg, 0)

    plsc.subcore_barrier()
    pltpu.sync_copy(acc.at[pl.ds(r0, RPT)], out.at[cid, pl.ds(r0, RPT)])
    if deg:
        pltpu.sync_copy(accd.at[pl.ds(r0, RPT)], outd.at[cid, pl.ds(r0, RPT)])


def _make_segsum(deg):
    def body(table_a, table_b, e0, e1, zrs, zrsd, ones, *refs):
        if deg:
            out, outd = refs[0], refs[1]
            rest = refs[2:]
        else:
            out, outd = refs[0], None
            rest = refs[1:]
        src_v, dst_v = rest[0], rest[1]
        rows = rest[2:2 + NR]
        ones_v, acc, accd = rest[2 + NR:5 + NR]
        sem_g = rest[5 + NR:5 + 2 * NR]
        sem_s = rest[5 + 2 * NR:5 + 3 * NR]
        sem_d = rest[5 + 3 * NR]
        cid = lax.axis_index("c")
        sid = lax.axis_index("s")
        args = (zrs, zrsd, ones, out, outd, cid, sid, src_v, dst_v,
                rows, ones_v, acc, accd, sem_g, sem_s, sem_d)

        @pl.when(cid == 0)
        def _():
            _direction(table_b, e0, *args, deg=deg)

        @pl.when(cid == 1)
        def _():
            _direction(table_a, e1, *args, deg=deg)

    out_type = [jax.ShapeDtypeStruct((NC, NP, D), jnp.bfloat16)]
    scratch = [
        pltpu.VMEM((NCH, CH), jnp.int32),
        pltpu.VMEM((NCH, CH), jnp.int32),
    ]
    scratch += [pltpu.VMEM((CH, D), jnp.bfloat16) for _ in range(NR)]
    scratch += [
        pltpu.VMEM((CH, DW), jnp.float32),
        pltpu.VMEM_SHARED((NP, D), jnp.bfloat16),
    ]
    if deg:
        out_type.append(jax.ShapeDtypeStruct((NC, NP, DW), jnp.float32))
        scratch.append(pltpu.VMEM_SHARED((NP, DW), jnp.float32))
    else:
        scratch.append(pltpu.VMEM((8, DW), jnp.float32))
    scratch += [pltpu.SemaphoreType.DMA] * (2 * NR + 1)

    return pl.kernel(
        body,
        mesh=_mesh,
        out_type=tuple(out_type),
        scratch_types=scratch,
        compiler_params=pltpu.CompilerParams(use_tc_tiling_on_sc=False),
    )


_segsum_deg = _make_segsum(deg=True)
_segsum = _make_segsum(deg=False)


_TCR = 1000


def _sage_tc_body(p, dg, x, wl, wr, b, scale, beta, out, *, lrelu):
    inv = 1.0 / jnp.maximum(dg[0][:, :1], 1.0)
    agg = p[0].astype(jnp.float32) * inv
    h = (lax.dot_general(agg, wl[...], _CONTRACT_LAST,
                         preferred_element_type=jnp.float32)
         + lax.dot_general(x[...].astype(jnp.float32), wr[...],
                           _CONTRACT_LAST,
                           preferred_element_type=jnp.float32)
         + b[...])
    h = h * scale[...] + beta[...]
    if lrelu:
        h = jnp.where(h >= 0.0, h, 0.01 * h)
    out[...] = h.astype(out.dtype)


def _make_tc(side, lrelu, xdtype, odtype):
    return pl.pallas_call(
        functools.partial(_sage_tc_body, lrelu=lrelu),
        grid=(N // _TCR,),
        in_specs=[
            pl.BlockSpec((1, _TCR, D), lambda i: (side, i, 0)),
            pl.BlockSpec((1, _TCR, DW), lambda i: (side, i, 0)),
            pl.BlockSpec((_TCR, D), lambda i: (i, 0)),
            pl.BlockSpec((D, D), lambda i: (0, 0)),
            pl.BlockSpec((D, D), lambda i: (0, 0)),
            pl.BlockSpec((1, D), lambda i: (0, 0)),
            pl.BlockSpec((1, D), lambda i: (0, 0)),
            pl.BlockSpec((1, D), lambda i: (0, 0)),
        ],
        out_specs=pl.BlockSpec((_TCR, D), lambda i: (i, 0)),
        out_shape=jax.ShapeDtypeStruct((N, D), odtype),
    )


_tc_user_l1 = _make_tc(0, True, jnp.float32, jnp.bfloat16)
_tc_item_l1 = _make_tc(1, True, jnp.float32, jnp.bfloat16)
_tc_user_l2 = _make_tc(0, False, jnp.bfloat16, jnp.float32)
_tc_item_l2 = _make_tc(1, False, jnp.bfloat16, jnp.float32)


def kernel(x_user, x_item, edge_index_rates, edge_index_rev_rates,
           W1l_ui, b1_ui, W1r_ui, W1l_iu, b1_iu, W1r_iu, gamma1, beta1,
           W2l_ui, b2_ui, W2r_ui, W2l_iu, b2_iu, W2r_iu, gamma2, beta2):
    dummy = jnp.concatenate(
        [jnp.zeros((1, EP - E), jnp.int32),
         jnp.full((1, EP - E), NP - 1, jnp.int32)])
    e0 = jnp.concatenate(
        [edge_index_rev_rates.astype(jnp.int32), dummy],
        axis=1).reshape(2, NS, NCH, CH)
    e1 = jnp.concatenate(
        [edge_index_rates.astype(jnp.int32), dummy],
        axis=1).reshape(2, NS, NCH, CH)
    xu = x_user.astype(jnp.bfloat16)
    xi = x_item.astype(jnp.bfloat16)
    zrs = jnp.zeros((NP, D), jnp.bfloat16)
    zrsd = jnp.zeros((NP, DW), jnp.float32)
    ones = jnp.ones((CH, DW), jnp.float32)

    bn = 1.0 / jnp.sqrt(1.0 + 1e-5)
    s1 = (gamma1 * bn).reshape(1, D)
    s2 = (gamma2 * bn).reshape(1, D)
    be1 = beta1.reshape(1, D)
    be2 = beta2.reshape(1, D)

    p, dg = _segsum_deg(xu, xi, e0, e1, zrs, zrsd, ones)
    h_user = _tc_user_l1(p, dg, x_user, W1l_iu, W1r_iu,
                         b1_iu.reshape(1, D), s1, be1)
    h_item = _tc_item_l1(p, dg, x_item, W1l_ui, W1r_ui,
                         b1_ui.reshape(1, D), s1, be1)

    (q,) = _segsum(h_user, h_item, e0, e1, zrs, zrsd, ones)
    o_user = _tc_user_l2(q, dg, h_user, W2l_iu, W2r_iu,
                         b2_iu.reshape(1, D), s2, be2)
    o_item = _tc_item_l2(q, dg, h_item, W2l_ui, W2r_ui,
                         b2_ui.reshape(1, D), s2, be2)
    return (o_user, o_item)


# speedup vs baseline: 1.9778x; 1.9778x over previous
import functools

import jax
import jax.numpy as jnp
from jax import lax
from jax.experimental import pallas as pl
from jax.experimental.pallas import tpu as pltpu
from jax.experimental.pallas import tpu_sc as plsc

N = 10000
D = 128
DW = 8
E = 320000
NC = 2
NS = 16
CH = 125
EPT = E // NS
NCH = EPT // CH
NR = 4
NP = 10240
RPT = NP // NS

_mesh = plsc.VectorSubcoreMesh(core_axis_name="c", subcore_axis_name="s")

_CONTRACT_LAST = (((1,), (1,)), ((), ()))


def _direction(table, edges, zrs, zrsd, ones, out, outd, cid, sid,
               src_v, dst_v, rows, ones_v, acc, accd,
               sem_g, sem_s, sem_d, deg):
    pltpu.sync_copy(edges.at[0, sid], src_v)
    pltpu.sync_copy(edges.at[1, sid], dst_v)
    r0 = sid * RPT
    pltpu.sync_copy(zrs.at[pl.ds(r0, RPT)], acc.at[pl.ds(r0, RPT)])
    if deg:
        pltpu.sync_copy(zrsd.at[pl.ds(r0, RPT)], accd.at[pl.ds(r0, RPT)])
        pltpu.sync_copy(ones, ones_v)
    plsc.subcore_barrier()

    def gather(c, u):
        pltpu.async_copy(table.at[src_v.at[c]], rows[u], sem_g[u])

    def drain_gather(u):
        pltpu.make_async_copy(table.at[src_v.at[0]], rows[u],
                              sem_g[u]).wait()

    def scatter(c, u):
        pltpu.async_copy(rows[u], acc.at[dst_v.at[c]], sem_s[u],
                         add=True)
        if deg:
            pltpu.async_copy(ones_v, accd.at[dst_v.at[c]], sem_d, add=True)

    def drain_scatter(u):
        pltpu.make_async_copy(rows[u], acc.at[dst_v.at[0]],
                              sem_s[u]).wait()

    for u in range(NR - 1):
        gather(u, u)

    def step(j, carry):
        for u in range(NR):
            c = j * NR + u
            v = (u + NR - 1) % NR
            drain_gather(u)
            scatter(c, u)

            @pl.when(c + NR - 1 < NCH)
            def _():
                @pl.when(c > 0)
                def _():
                    drain_scatter(v)
                gather(c + NR - 1, v)

        return carry

    lax.fori_loop(0, NCH // NR, step, 0)

    for u in range(NR):
        drain_scatter(u)
    if deg:
        def drain_deg(i, carry):
            pltpu.make_async_copy(ones_v, accd.at[dst_v.at[0]], sem_d).wait()
            return carry
        lax.fori_loop(0, NCH, drain_deg, 0)

    plsc.subcore_barrier()
    pltpu.sync_copy(acc.at[pl.ds(r0, RPT)], out.at[cid, pl.ds(r0, RPT)])
    if deg:
        pltpu.sync_copy(accd.at[pl.ds(r0, RPT)], outd.at[cid, pl.ds(r0, RPT)])


def _make_segsum(deg):
    def body(table_a, table_b, e0, e1, zrs, zrsd, ones, *refs):
        if deg:
            out, outd = refs[0], refs[1]
            rest = refs[2:]
        else:
            out, outd = refs[0], None
            rest = refs[1:]
        src_v, dst_v = rest[0], rest[1]
        rows = rest[2:2 + NR]
        ones_v, acc, accd = rest[2 + NR:5 + NR]
        sem_g = rest[5 + NR:5 + 2 * NR]
        sem_s = rest[5 + 2 * NR:5 + 3 * NR]
        sem_d = rest[5 + 3 * NR]
        cid = lax.axis_index("c")
        sid = lax.axis_index("s")
        args = (zrs, zrsd, ones, out, outd, cid, sid, src_v, dst_v,
                rows, ones_v, acc, accd, sem_g, sem_s, sem_d)

        @pl.when(cid == 0)
        def _():
            _direction(table_b, e0, *args, deg=deg)

        @pl.when(cid == 1)
        def _():
            _direction(table_a, e1, *args, deg=deg)

    out_type = [jax.ShapeDtypeStruct((NC, NP, D), jnp.bfloat16)]
    scratch = [
        pltpu.VMEM((NCH, CH), jnp.int32),
        pltpu.VMEM((NCH, CH), jnp.int32),
    ]
    scratch += [pltpu.VMEM((CH, D), jnp.bfloat16) for _ in range(NR)]
    scratch += [
        pltpu.VMEM((CH, DW), jnp.float32),
        pltpu.VMEM_SHARED((NP, D), jnp.bfloat16),
    ]
    if deg:
        out_type.append(jax.ShapeDtypeStruct((NC, NP, DW), jnp.float32))
        scratch.append(pltpu.VMEM_SHARED((NP, DW), jnp.float32))
    else:
        scratch.append(pltpu.VMEM((8, DW), jnp.float32))
    scratch += [pltpu.SemaphoreType.DMA] * (2 * NR + 1)

    return pl.kernel(
        body,
        mesh=_mesh,
        out_type=tuple(out_type),
        scratch_types=scratch,
        compiler_params=pltpu.CompilerParams(use_tc_tiling_on_sc=False),
    )


_segsum_deg = _make_segsum(deg=True)
_segsum = _make_segsum(deg=False)


_TCR = 1024
_FB = _TCR * D


def _sage_tc_body(p, dg, x, wl, wr, b, scale, beta, out, *, lrelu):
    inv = 1.0 / jnp.maximum(dg[0][:, :1], 1.0)
    agg = p[...].reshape(_TCR, D).astype(jnp.float32) * inv
    xb = x[...].reshape(_TCR, D).astype(jnp.float32)
    h = (lax.dot_general(agg, wl[...], _CONTRACT_LAST,
                         preferred_element_type=jnp.float32)
         + lax.dot_general(xb, wr[...], _CONTRACT_LAST,
                           preferred_element_type=jnp.float32)
         + b[...])
    h = h * scale[...] + beta[...]
    if lrelu:
        h = jnp.where(h >= 0.0, h, 0.01 * h)
    if out.dtype == jnp.bfloat16:
        out[...] = h.astype(jnp.bfloat16).reshape(_FB)
    else:
        out[...] = h


def _make_tc(side, lrelu, odtype):
    if odtype == jnp.bfloat16:
        out_spec = pl.BlockSpec((_FB,), lambda i: (i,))
        out_shape = jax.ShapeDtypeStruct((NP * D,), jnp.bfloat16)
    else:
        out_spec = pl.BlockSpec((_TCR, D), lambda i: (i, 0))
        out_shape = jax.ShapeDtypeStruct((NP, D), jnp.float32)
    return pl.pallas_call(
        functools.partial(_sage_tc_body, lrelu=lrelu),
        grid=(NP // _TCR,),
        in_specs=[
            pl.BlockSpec((_FB,), lambda i: (side * (NP // _TCR) + i,)),
            pl.BlockSpec((1, _TCR, DW), lambda i: (side, i, 0)),
            pl.BlockSpec((_FB,), lambda i: (i,)),
            pl.BlockSpec((D, D), lambda i: (0, 0)),
            pl.BlockSpec((D, D), lambda i: (0, 0)),
            pl.BlockSpec((1, D), lambda i: (0, 0)),
            pl.BlockSpec((1, D), lambda i: (0, 0)),
            pl.BlockSpec((1, D), lambda i: (0, 0)),
        ],
        out_specs=out_spec,
        out_shape=out_shape,
    )


_tc_user_l1 = _make_tc(0, True, jnp.bfloat16)
_tc_item_l1 = _make_tc(1, True, jnp.bfloat16)
_tc_user_l2 = _make_tc(0, False, jnp.float32)
_tc_item_l2 = _make_tc(1, False, jnp.float32)


def kernel(x_user, x_item, edge_index_rates, edge_index_rev_rates,
           W1l_ui, b1_ui, W1r_ui, W1l_iu, b1_iu, W1r_iu, gamma1, beta1,
           W2l_ui, b2_ui, W2r_ui, W2l_iu, b2_iu, W2r_iu, gamma2, beta2):
    e0 = edge_index_rev_rates.astype(jnp.int32).reshape(2, NS, NCH, CH)
    e1 = edge_index_rates.astype(jnp.int32).reshape(2, NS, NCH, CH)
    pad = ((0, NP - N), (0, 0))
    xu = jnp.pad(x_user, pad).astype(jnp.bfloat16)
    xi = jnp.pad(x_item, pad).astype(jnp.bfloat16)
    zrs = jnp.zeros((NP, D), jnp.bfloat16)
    zrsd = jnp.zeros((NP, DW), jnp.float32)
    ones = jnp.ones((CH, DW), jnp.float32)

    bn = 1.0 / jnp.sqrt(1.0 + 1e-5)
    s1 = (gamma1 * bn).reshape(1, D)
    s2 = (gamma2 * bn).reshape(1, D)
    be1 = beta1.reshape(1, D)
    be2 = beta2.reshape(1, D)

    p, dg = _segsum_deg(xu, xi, e0, e1, zrs, zrsd, ones)
    p1d = p.reshape(NC * NP * D)
    h_user = _tc_user_l1(p1d, dg, xu.reshape(NP * D), W1l_iu, W1r_iu,
                         b1_iu.reshape(1, D), s1, be1)
    h_item = _tc_item_l1(p1d, dg, xi.reshape(NP * D), W1l_ui, W1r_ui,
                         b1_ui.reshape(1, D), s1, be1)

    (q,) = _segsum(h_user.reshape(NP, D), h_item.reshape(NP, D),
                   e0, e1, zrs, zrsd, ones)
    q1d = q.reshape(NC * NP * D)
    o_user = _tc_user_l2(q1d, dg, h_user, W2l_iu, W2r_iu,
                         b2_iu.reshape(1, D), s2, be2)
    o_item = _tc_item_l2(q1d, dg, h_item, W2l_ui, W2r_ui,
                         b2_ui.reshape(1, D), s2, be2)
    return (o_user[:N], o_item[:N])
